# baseline (speedup 1.0000x reference)
import math

import jax
import jax.numpy as jnp
from jax.experimental import pallas as pl

_N = 20000
_NPAD = 20480
_NROWS = 160
_K1 = 2000
_KPAD = 2048
_NBLK = 16
_B = 128
_K2 = 1000
_T = 0.7
_CLAMP = math.log(1000.0 / 16.0)
_IMG = 1024.0


def _dot(a, b):
    return jax.lax.dot_general(a, b, (((1,), (0,)), ((), ())),
                               preferred_element_type=jnp.float32)


def _dotx(a, b):
    return jax.lax.dot_general(a, b, (((1,), (0,)), ((), ())),
                               preferred_element_type=jnp.float32,
                               precision=jax.lax.Precision.HIGHEST)


def _decode_body(ax1, ay1, ax2, ay2, dx, dy, dw, dh, sc,
                 ox1, oy1, ox2, oy2, os):
    w = ax2[...] - ax1[...]
    h = ay2[...] - ay1[...]
    cx = ax1[...] + 0.5 * w
    cy = ay1[...] + 0.5 * h
    ddw = jnp.minimum(dw[...], _CLAMP)
    ddh = jnp.minimum(dh[...], _CLAMP)
    px = dx[...] * w + cx
    py = dy[...] * h + cy
    pw = jnp.exp(ddw) * w
    ph = jnp.exp(ddh) * h
    x1 = jnp.clip(px - 0.5 * pw, 0.0, _IMG)
    y1 = jnp.clip(py - 0.5 * ph, 0.0, _IMG)
    x2 = jnp.clip(px + 0.5 * pw, 0.0, _IMG)
    y2 = jnp.clip(py + 0.5 * ph, 0.0, _IMG)
    valid = ((x2 - x1) > 0.0) & ((y2 - y1) > 0.0)
    ox1[...] = x1
    oy1[...] = y1
    ox2[...] = x2
    oy2[...] = y2
    os[...] = jnp.where(valid, sc[...], -jnp.inf)


def _nms_body(x1c, y1c, x2c, y2c, x1r, y1r, x2r, y2r, vr, out40):
    cx1, cy1, cx2, cy2 = x1c[...], y1c[...], x2c[...], y2c[...]
    area_c = (cx2 - cx1) * (cy2 - cy1)
    colg = jax.lax.broadcasted_iota(jnp.int32, (1, _KPAD), 1)
    keep = colg < _K1

    li = jax.lax.broadcasted_iota(jnp.int32, (_B, _B), 0)
    lj = jax.lax.broadcasted_iota(jnp.int32, (_B, _B), 1)
    upper = li < lj

    for r in range(_NBLK):
        rx1 = x1r[:, r:r + 1]
        ry1 = y1r[:, r:r + 1]
        rx2 = x2r[:, r:r + 1]
        ry2 = y2r[:, r:r + 1]
        area_r = (rx2 - rx1) * (ry2 - ry1)
        wx = jnp.maximum(jnp.minimum(rx2, cx2) - jnp.maximum(rx1, cx1), 0.0)
        wy = jnp.maximum(jnp.minimum(ry2, cy2) - jnp.maximum(ry1, cy1), 0.0)
        inter = wx * wy
        union = area_r + area_c - inter
        mf = (inter > _T * jnp.maximum(union, 1e-9)).astype(jnp.float32)
        mbb = jnp.where(upper, mf[:, r * _B:(r + 1) * _B], 0.0)

        active0 = keep[:, r * _B:(r + 1) * _B].astype(jnp.float32)

        def cond(c):
            return jnp.logical_not(c[2])

        def body(c):
            act, _, _ = c
            pse = _dot(act, mbb) > 0.0
            knew = jnp.where(pse, 0.0, act)
            ns = _dot(knew, mbb) > 0.0
            act2 = jnp.where(ns, 0.0, act)
            done = jnp.all(act2 == act)
            return act2, knew, done

        _, kept, _ = jax.lax.while_loop(
            cond, body, (active0, active0, jnp.bool_(False)))

        supp_later = (_dot(kept, mf) > 0.0) & (colg >= (r + 1) * _B)
        su_blk = (active0 > 0.0) & jnp.logical_not(kept > 0.0)
        parts = []
        if r > 0:
            parts.append(jnp.zeros((1, r * _B), jnp.bool_))
        parts.append(su_blk)
        if r < _NBLK - 1:
            parts.append(jnp.zeros((1, (_NBLK - 1 - r) * _B), jnp.bool_))
        su_full = jnp.concatenate(parts, axis=1)
        keep = keep & jnp.logical_not(supp_later | su_full)

    kept16 = jnp.reshape(keep.astype(jnp.float32), (_NBLK, _B))
    lidx = jax.lax.broadcasted_iota(jnp.int32, (_B, _B), 0)
    lcol = jax.lax.broadcasted_iota(jnp.int32, (_B, _B), 1)
    tu = (lidx <= lcol).astype(jnp.float32)
    bi = jax.lax.broadcasted_iota(jnp.int32, (_NBLK, _NBLK), 0)
    bj = jax.lax.broadcasted_iota(jnp.int32, (_NBLK, _NBLK), 1)
    s16 = (bj < bi).astype(jnp.float32)

    def _explusive_prefix(m16):
        inc = _dot(m16, tu)
        rt = inc[:, _B - 1:_B]
        off = _dot(s16, rt)
        return inc - m16 + off

    exk = _explusive_prefix(kept16)
    nk16 = 1.0 - kept16
    exn = _explusive_prefix(nk16)
    ktot = _dot(jnp.ones((1, _NBLK), jnp.float32),
                _dot(kept16, tu)[:, _B - 1:_B])
    pos2f = jnp.where(kept16 > 0.0, exk, ktot + exn)
    pos2 = pos2f.astype(jnp.int32)
    lo16 = pos2 & (_B - 1)
    hi16 = pos2 >> 7
    valid16 = pos2 < 1024
    hi_rows = jnp.transpose(pos2f).astype(jnp.int32) >> 7

    iota_cl = jax.lax.broadcasted_iota(jnp.int32, (_B, _B), 0)
    iota_l = jax.lax.broadcasted_iota(jnp.int32, (_B, _B), 1)
    iota_h8 = jax.lax.broadcasted_iota(jnp.int32, (_B, 8), 1)
    acc = jnp.zeros((_B, 40), jnp.float32)
    for r in range(_NBLK):
        lo_row = lo16[r:r + 1, :]
        v_row = valid16[r:r + 1, :]
        ohlo_t = ((iota_cl == lo_row) & v_row).astype(jnp.float32)
        hi_col = hi_rows[:, r:r + 1]
        ohhi = (hi_col == iota_h8).astype(jnp.float32)
        dall = jnp.concatenate(
            [f[:, r:r + 1] * ohhi for f in (x1r, y1r, x2r, y2r, vr)], axis=1)
        acc = acc + _dotx(ohlo_t, dall)
    out40[...] = acc


def _decode(planes):
    outs = [jax.ShapeDtypeStruct((_NROWS, 128), jnp.float32)] * 5
    return pl.pallas_call(_decode_body, out_shape=outs)(*planes)


def _nms(cols, rows, vr):
    return pl.pallas_call(
        _nms_body,
        out_shape=jax.ShapeDtypeStruct((_B, 40), jnp.float32),
    )(*cols, *rows, vr)


def kernel(anchors, deltas, scores):
    a = jnp.pad(anchors, ((0, _NPAD - _N), (0, 0)))
    d = jnp.pad(deltas, ((0, _NPAD - _N), (0, 0)))
    s = jnp.pad(scores, (0, _NPAD - _N))
    planes = [a[:, i].reshape(_NROWS, 128) for i in range(4)]
    planes += [d[:, i].reshape(_NROWS, 128) for i in range(4)]
    planes.append(s.reshape(_NROWS, 128))
    x1, y1, x2, y2, sm = _decode(planes)

    vals, idx = jax.lax.top_k(sm.reshape(-1), _K1)
    g = [f.reshape(-1)[idx] for f in (x1, y1, x2, y2)]

    pad = _KPAD - _K1
    cols = [jnp.pad(f, (0, pad)).reshape(1, _KPAD) for f in g]
    rows = [jnp.pad(f, (0, pad)).reshape(_NBLK, _B).T for f in g]
    vr = jnp.maximum(jnp.pad(vals, (0, pad), constant_values=-jnp.inf),
                     jnp.float32(-3.4e38)).reshape(_NBLK, _B).T
    out40 = _nms(cols, rows, vr)

    fields = [out40[:, 8 * f:8 * (f + 1)].T.reshape(-1)[:_K2]
              for f in range(5)]
    return jnp.stack(fields, axis=1)


# speedup vs baseline: 1.0849x; 1.0849x over previous
import math

import jax
import jax.numpy as jnp
from jax.experimental import pallas as pl

_N = 20000
_NPAD = 20480
_NROWS = 160
_K1 = 2000
_KPAD = 2048
_NBLK = 16
_B = 128
_K2 = 1000
_T = 0.7
_CLAMP = math.log(1000.0 / 16.0)
_IMG = 1024.0
_NEG = -3.4e38
_IMIN = -2147483648


def _dot(a, b):
    return jax.lax.dot_general(a, b, (((1,), (0,)), ((), ())),
                               preferred_element_type=jnp.float32)


def _dotx(a, b):
    return jax.lax.dot_general(a, b, (((1,), (0,)), ((), ())),
                               preferred_element_type=jnp.float32,
                               precision=jax.lax.Precision.HIGHEST)


def _iota(shape, dim):
    return jax.lax.broadcasted_iota(jnp.int32, shape, dim)


def _exclusive_prefix(m):
    tu = (_iota((_B, _B), 0) <= _iota((_B, _B), 1)).astype(jnp.float32)
    s_r = (_iota((_NROWS, _NROWS), 1) < _iota((_NROWS, _NROWS), 0))
    inc = _dot(m, tu)
    rt = inc[:, _B - 1:_B]
    off = _dot(s_r.astype(jnp.float32), rt)
    return inc - m + off


def _main_body(ax1, ay1, ax2, ay2, dx, dy, dw, dh, sc, out40):
    w = ax2[...] - ax1[...]
    h = ay2[...] - ay1[...]
    cx = ax1[...] + 0.5 * w
    cy = ay1[...] + 0.5 * h
    ddw = jnp.minimum(dw[...], _CLAMP)
    ddh = jnp.minimum(dh[...], _CLAMP)
    px = dx[...] * w + cx
    py = dy[...] * h + cy
    pw = jnp.exp(ddw) * w
    ph = jnp.exp(ddh) * h
    bx1 = jnp.clip(px - 0.5 * pw, 0.0, _IMG)
    by1 = jnp.clip(py - 0.5 * ph, 0.0, _IMG)
    bx2 = jnp.clip(px + 0.5 * pw, 0.0, _IMG)
    by2 = jnp.clip(py + 0.5 * ph, 0.0, _IMG)
    valid = ((bx2 - bx1) > 0.0) & ((by2 - by1) > 0.0)
    sv = jnp.where(valid, jnp.maximum(sc[...], _NEG), _NEG)

    ib = jax.lax.bitcast_convert_type(sv, jnp.int32)
    key = jnp.where(ib >= 0, ib, ib ^ jnp.int32(0x7FFFFFFF))
    ukey = key ^ jnp.int32(_IMIN)

    def rbody(t, c):
        cand, prefix, r = c
        k = 31 - t
        bit = jax.lax.shift_left(jnp.int32(1), k)
        hasf = jnp.where((ukey & bit) != 0, 1.0, 0.0)
        c1 = jnp.sum(cand * hasf)
        take = c1 >= r
        cand2 = jnp.where(take, cand * hasf, cand * (1.0 - hasf))
        prefix2 = jnp.where(take, prefix | bit, prefix)
        r2 = jnp.where(take, r, r - c1)
        return cand2, prefix2, r2

    ones = jnp.ones((_NROWS, _B), jnp.float32)
    tief, vu, r_fin = jax.lax.fori_loop(
        0, 32, rbody, (ones, jnp.int32(0), jnp.float32(_K1)))
    vk = vu ^ _IMIN
    tie = tief > 0.0

    tie_pre = _exclusive_prefix(tief)
    sel = (key > vk) | (tie & (tie_pre < r_fin))
    self32 = jnp.where(sel, 1.0, 0.0)
    pos = _exclusive_prefix(self32)

    posi = pos.astype(jnp.int32)
    lo = posi & (_B - 1)
    posT = jnp.transpose(pos)
    hiT = posT.astype(jnp.int32) >> 7
    dataT = [jnp.transpose(f) for f in (bx1, by1, bx2, by2, sv)]
    iota_cl = _iota((_B, _B), 0)
    iota_h16 = _iota((_B, _NBLK), 1)
    acc_c = jnp.zeros((_B, 80), jnp.float32)
    for i in range(_NROWS):
        lo_row = lo[i:i + 1, :]
        sel_row = sel[i:i + 1, :]
        ohlo_t = ((iota_cl == lo_row) & sel_row).astype(jnp.float32)
        ohhi = (hiT[:, i:i + 1] == iota_h16).astype(jnp.float32)
        dall = jnp.concatenate([f[:, i:i + 1] * ohhi for f in dataT], axis=1)
        acc_c = acc_c + _dotx(ohlo_t, dall)

    cfields = [acc_c[:, 16 * f:16 * (f + 1)] for f in range(5)]
    slot_rows = _iota((_B, _NBLK), 0) + _B * _iota((_B, _NBLK), 1)
    csv = jnp.where(slot_rows < _K1, cfields[4], _NEG)

    csv_cols = jnp.reshape(jnp.transpose(csv), (1, _KPAD))
    slot_cols = _iota((1, _KPAD), 1)
    ord_parts = []
    for r in range(_NBLK):
        kb = csv[:, r:r + 1]
        sb = slot_rows[:, r:r + 1]
        gt = jnp.sum((csv_cols > kb).astype(jnp.float32), axis=1,
                     keepdims=True)
        eqlt = jnp.sum(((csv_cols == kb) & (slot_cols < sb)).astype(
            jnp.float32), axis=1, keepdims=True)
        ord_parts.append(gt + eqlt)
    ordr = jnp.concatenate(ord_parts, axis=1)

    ordT = jnp.transpose(ordr)
    acc_s = jnp.zeros((_B, 80), jnp.float32)
    for r in range(_NBLK):
        lo_row = ordT[r:r + 1, :].astype(jnp.int32) & (_B - 1)
        ohlo_t = (iota_cl == lo_row).astype(jnp.float32)
        hi_col = ordr[:, r:r + 1].astype(jnp.int32) >> 7
        ohhi = (hi_col == iota_h16).astype(jnp.float32)
        dall = jnp.concatenate(
            [f[:, r:r + 1] * ohhi for f in cfields], axis=1)
        acc_s = acc_s + _dotx(ohlo_t, dall)
    x1r, y1r, x2r, y2r, vr = [acc_s[:, 16 * f:16 * (f + 1)] for f in range(5)]

    cx1, cy1, cx2, cy2 = [jnp.reshape(jnp.transpose(f), (1, _KPAD))
                          for f in (x1r, y1r, x2r, y2r)]
    area_c = (cx2 - cx1) * (cy2 - cy1)
    colg = _iota((1, _KPAD), 1)
    keep = colg < _K1

    upper = _iota((_B, _B), 0) < _iota((_B, _B), 1)

    for r in range(_NBLK):
        rx1 = x1r[:, r:r + 1]
        ry1 = y1r[:, r:r + 1]
        rx2 = x2r[:, r:r + 1]
        ry2 = y2r[:, r:r + 1]
        area_r = (rx2 - rx1) * (ry2 - ry1)
        wx = jnp.maximum(jnp.minimum(rx2, cx2) - jnp.maximum(rx1, cx1), 0.0)
        wy = jnp.maximum(jnp.minimum(ry2, cy2) - jnp.maximum(ry1, cy1), 0.0)
        inter = wx * wy
        union = area_r + area_c - inter
        mf = (inter > _T * jnp.maximum(union, 1e-9)).astype(jnp.float32)
        mbb = jnp.where(upper, mf[:, r * _B:(r + 1) * _B], 0.0)

        active0 = keep[:, r * _B:(r + 1) * _B].astype(jnp.float32)

        def cond(c):
            return jnp.logical_not(c[2])

        def body(c):
            act, _, _ = c
            pse = _dot(act, mbb) > 0.0
            knew = jnp.where(pse, 0.0, act)
            ns = _dot(knew, mbb) > 0.0
            act2 = jnp.where(ns, 0.0, act)
            done = jnp.all(act2 == act)
            return act2, knew, done

        _, kept, _ = jax.lax.while_loop(
            cond, body, (active0, active0, jnp.bool_(False)))

        supp_later = (_dot(kept, mf) > 0.0) & (colg >= (r + 1) * _B)
        su_blk = (active0 > 0.0) & jnp.logical_not(kept > 0.0)
        parts = []
        if r > 0:
            parts.append(jnp.zeros((1, r * _B), jnp.bool_))
        parts.append(su_blk)
        if r < _NBLK - 1:
            parts.append(jnp.zeros((1, (_NBLK - 1 - r) * _B), jnp.bool_))
        su_full = jnp.concatenate(parts, axis=1)
        keep = keep & jnp.logical_not(supp_later | su_full)

    kept16 = jnp.reshape(keep.astype(jnp.float32), (_NBLK, _B))
    tu = (_iota((_B, _B), 0) <= _iota((_B, _B), 1)).astype(jnp.float32)
    s16 = (_iota((_NBLK, _NBLK), 1) < _iota((_NBLK, _NBLK), 0)).astype(
        jnp.float32)

    def _explusive_prefix16(m16):
        inc = _dot(m16, tu)
        rt = inc[:, _B - 1:_B]
        off = _dot(s16, rt)
        return inc - m16 + off

    exk = _explusive_prefix16(kept16)
    nk16 = 1.0 - kept16
    exn = _explusive_prefix16(nk16)
    ktot = _dot(jnp.ones((1, _NBLK), jnp.float32),
                _dot(kept16, tu)[:, _B - 1:_B])
    pos2f = jnp.where(kept16 > 0.0, exk, ktot + exn)
    pos2 = pos2f.astype(jnp.int32)
    lo16 = pos2 & (_B - 1)
    valid16 = pos2 < 1024
    hi_rows = jnp.transpose(pos2f).astype(jnp.int32) >> 7

    iota_h8 = _iota((_B, 8), 1)
    acc = jnp.zeros((_B, 40), jnp.float32)
    for r in range(_NBLK):
        lo_row = lo16[r:r + 1, :]
        v_row = valid16[r:r + 1, :]
        ohlo_t = ((iota_cl == lo_row) & v_row).astype(jnp.float32)
        hi_col = hi_rows[:, r:r + 1]
        ohhi = (hi_col == iota_h8).astype(jnp.float32)
        dall = jnp.concatenate(
            [f[:, r:r + 1] * ohhi for f in (x1r, y1r, x2r, y2r, vr)], axis=1)
        acc = acc + _dotx(ohlo_t, dall)
    out40[...] = acc


def kernel(anchors, deltas, scores):
    a = jnp.pad(anchors, ((0, _NPAD - _N), (0, 0)))
    d = jnp.pad(deltas, ((0, _NPAD - _N), (0, 0)))
    s = jnp.pad(scores, (0, _NPAD - _N))
    planes = [a[:, i].reshape(_NROWS, _B) for i in range(4)]
    planes += [d[:, i].reshape(_NROWS, _B) for i in range(4)]
    planes.append(s.reshape(_NROWS, _B))
    out40 = pl.pallas_call(
        _main_body,
        out_shape=jax.ShapeDtypeStruct((_B, 40), jnp.float32),
    )(*planes)
    fields = [out40[:, 8 * f:8 * (f + 1)].T.reshape(-1)[:_K2]
              for f in range(5)]
    return jnp.stack(fields, axis=1)


# speedup vs baseline: 1.1004x; 1.0143x over previous
import math

import jax
import jax.numpy as jnp
from jax.experimental import pallas as pl

_N = 20000
_NPAD = 20480
_NROWS = 160
_K1 = 2000
_KPAD = 2048
_NBLK = 16
_B = 128
_K2 = 1000
_T = 0.7
_CLAMP = math.log(1000.0 / 16.0)
_IMG = 1024.0
_NEG = -3.4e38
_IMIN = -2147483648


def _dot(a, b):
    return jax.lax.dot_general(a, b, (((1,), (0,)), ((), ())),
                               preferred_element_type=jnp.float32)


def _dotx(a, b):
    return jax.lax.dot_general(a, b, (((1,), (0,)), ((), ())),
                               preferred_element_type=jnp.float32,
                               precision=jax.lax.Precision.HIGHEST)


def _iota(shape, dim):
    return jax.lax.broadcasted_iota(jnp.int32, shape, dim)


def _exclusive_prefix(m):
    tu = (_iota((_B, _B), 0) <= _iota((_B, _B), 1)).astype(jnp.float32)
    s_r = (_iota((_NROWS, _NROWS), 1) < _iota((_NROWS, _NROWS), 0))
    inc = _dot(m, tu)
    rt = inc[:, _B - 1:_B]
    off = _dot(s_r.astype(jnp.float32), rt)
    return inc - m + off


def _main_body(ax1, ay1, ax2, ay2, dx, dy, dw, dh, sc, out40):
    w = ax2[...] - ax1[...]
    h = ay2[...] - ay1[...]
    cx = ax1[...] + 0.5 * w
    cy = ay1[...] + 0.5 * h
    ddw = jnp.minimum(dw[...], _CLAMP)
    ddh = jnp.minimum(dh[...], _CLAMP)
    px = dx[...] * w + cx
    py = dy[...] * h + cy
    pw = jnp.exp(ddw) * w
    ph = jnp.exp(ddh) * h
    bx1 = jnp.clip(px - 0.5 * pw, 0.0, _IMG)
    by1 = jnp.clip(py - 0.5 * ph, 0.0, _IMG)
    bx2 = jnp.clip(px + 0.5 * pw, 0.0, _IMG)
    by2 = jnp.clip(py + 0.5 * ph, 0.0, _IMG)
    valid = ((bx2 - bx1) > 0.0) & ((by2 - by1) > 0.0)
    sv = jnp.where(valid, jnp.maximum(sc[...], _NEG), _NEG)

    ib = jax.lax.bitcast_convert_type(sv, jnp.int32)
    key = jnp.where(ib >= 0, ib, ib ^ jnp.int32(0x7FFFFFFF))
    ukey = key ^ jnp.int32(_IMIN)

    def rbody(t, c):
        cand, prefix, r = c
        k = 31 - t
        bit = jax.lax.shift_left(jnp.int32(1), k)
        hasf = jnp.where((ukey & bit) != 0, 1.0, 0.0)
        c1 = jnp.sum(cand * hasf)
        take = c1 >= r
        cand2 = jnp.where(take, cand * hasf, cand * (1.0 - hasf))
        prefix2 = jnp.where(take, prefix | bit, prefix)
        r2 = jnp.where(take, r, r - c1)
        return cand2, prefix2, r2

    ones = jnp.ones((_NROWS, _B), jnp.float32)
    tief, vu, r_fin = jax.lax.fori_loop(
        0, 32, rbody, (ones, jnp.int32(0), jnp.float32(_K1)))
    vk = vu ^ _IMIN
    tie = tief > 0.0

    tie_pre = _exclusive_prefix(tief)
    sel = (key > vk) | (tie & (tie_pre < r_fin))
    self32 = jnp.where(sel, 1.0, 0.0)
    pos = _exclusive_prefix(self32)

    posi = pos.astype(jnp.int32)
    lo = jnp.where(sel, posi & (_B - 1), _B)
    posT = jnp.transpose(pos)
    hiT = posT.astype(jnp.int32) >> 7
    dataT = [jnp.transpose(f) for f in (bx1, by1, bx2, by2, sv)]
    iota_cl = _iota((_B, _B), 0)
    iota_h16 = _iota((_B, _NBLK), 1)
    acc_c = jnp.zeros((_B, 80), jnp.float32)
    for i in range(_NROWS):
        lo_row = lo[i:i + 1, :]
        ohlo_t = (iota_cl == lo_row).astype(jnp.float32)
        ohhi = (hiT[:, i:i + 1] == iota_h16).astype(jnp.float32)
        dall = jnp.concatenate([f[:, i:i + 1] * ohhi for f in dataT], axis=1)
        acc_c = acc_c + _dotx(ohlo_t, dall)

    cfields = [acc_c[:, 16 * f:16 * (f + 1)] for f in range(5)]
    slot_rows = _iota((_B, _NBLK), 0) + _B * _iota((_B, _NBLK), 1)
    csv = jnp.where(slot_rows < _K1, cfields[4], _NEG)

    csv_cols = jnp.reshape(jnp.transpose(csv), (1, _KPAD))
    slot_cols = _iota((1, _KPAD), 1)
    ord_parts = []
    for r in range(_NBLK):
        kb = csv[:, r:r + 1]
        sb = slot_rows[:, r:r + 1]
        gt = jnp.sum((csv_cols > kb).astype(jnp.float32), axis=1,
                     keepdims=True)
        eqlt = jnp.sum(((csv_cols == kb) & (slot_cols < sb)).astype(
            jnp.float32), axis=1, keepdims=True)
        ord_parts.append(gt + eqlt)
    ordr = jnp.concatenate(ord_parts, axis=1)

    ordT = jnp.transpose(ordr)
    acc_s = jnp.zeros((_B, 80), jnp.float32)
    for r in range(_NBLK):
        lo_row = ordT[r:r + 1, :].astype(jnp.int32) & (_B - 1)
        ohlo_t = (iota_cl == lo_row).astype(jnp.float32)
        hi_col = ordr[:, r:r + 1].astype(jnp.int32) >> 7
        ohhi = (hi_col == iota_h16).astype(jnp.float32)
        dall = jnp.concatenate(
            [f[:, r:r + 1] * ohhi for f in cfields], axis=1)
        acc_s = acc_s + _dotx(ohlo_t, dall)
    x1r, y1r, x2r, y2r, vr = [acc_s[:, 16 * f:16 * (f + 1)] for f in range(5)]

    cx1, cy1, cx2, cy2 = [jnp.reshape(jnp.transpose(f), (1, _KPAD))
                          for f in (x1r, y1r, x2r, y2r)]
    area_c = (cx2 - cx1) * (cy2 - cy1)
    colg = _iota((1, _KPAD), 1)
    keep = colg < _K1

    upper = _iota((_B, _B), 0) < _iota((_B, _B), 1)

    for r in range(_NBLK):
        rx1 = x1r[:, r:r + 1]
        ry1 = y1r[:, r:r + 1]
        rx2 = x2r[:, r:r + 1]
        ry2 = y2r[:, r:r + 1]
        area_r = (rx2 - rx1) * (ry2 - ry1)
        wx = jnp.maximum(jnp.minimum(rx2, cx2) - jnp.maximum(rx1, cx1), 0.0)
        wy = jnp.maximum(jnp.minimum(ry2, cy2) - jnp.maximum(ry1, cy1), 0.0)
        inter = wx * wy
        union = area_r + area_c - inter
        mf = (inter > _T * jnp.maximum(union, 1e-9)).astype(jnp.float32)
        mbb = jnp.where(upper, mf[:, r * _B:(r + 1) * _B], 0.0)

        active0 = keep[:, r * _B:(r + 1) * _B].astype(jnp.float32)

        def cond(c):
            return jnp.logical_not(c[2])

        def body(c):
            act, _, _ = c
            pse = _dot(act, mbb) > 0.0
            knew = jnp.where(pse, 0.0, act)
            ns = _dot(knew, mbb) > 0.0
            act2 = jnp.where(ns, 0.0, act)
            done = jnp.all(act2 == act)
            return act2, knew, done

        _, kept, _ = jax.lax.while_loop(
            cond, body, (active0, active0, jnp.bool_(False)))

        supp_later = (_dot(kept, mf) > 0.0) & (colg >= (r + 1) * _B)
        su_blk = (active0 > 0.0) & jnp.logical_not(kept > 0.0)
        parts = []
        if r > 0:
            parts.append(jnp.zeros((1, r * _B), jnp.bool_))
        parts.append(su_blk)
        if r < _NBLK - 1:
            parts.append(jnp.zeros((1, (_NBLK - 1 - r) * _B), jnp.bool_))
        su_full = jnp.concatenate(parts, axis=1)
        keep = keep & jnp.logical_not(supp_later | su_full)

    kept16 = jnp.reshape(keep.astype(jnp.float32), (_NBLK, _B))
    tu = (_iota((_B, _B), 0) <= _iota((_B, _B), 1)).astype(jnp.float32)
    s16 = (_iota((_NBLK, _NBLK), 1) < _iota((_NBLK, _NBLK), 0)).astype(
        jnp.float32)

    def _explusive_prefix16(m16):
        inc = _dot(m16, tu)
        rt = inc[:, _B - 1:_B]
        off = _dot(s16, rt)
        return inc - m16 + off

    exk = _explusive_prefix16(kept16)
    nk16 = 1.0 - kept16
    exn = _explusive_prefix16(nk16)
    ktot = _dot(jnp.ones((1, _NBLK), jnp.float32),
                _dot(kept16, tu)[:, _B - 1:_B])
    pos2f = jnp.where(kept16 > 0.0, exk, ktot + exn)
    pos2 = pos2f.astype(jnp.int32)
    lo16 = pos2 & (_B - 1)
    valid16 = pos2 < 1024
    hi_rows = jnp.transpose(pos2f).astype(jnp.int32) >> 7

    iota_h8 = _iota((_B, 8), 1)
    acc = jnp.zeros((_B, 40), jnp.float32)
    for r in range(_NBLK):
        lo_row = lo16[r:r + 1, :]
        v_row = valid16[r:r + 1, :]
        ohlo_t = ((iota_cl == lo_row) & v_row).astype(jnp.float32)
        hi_col = hi_rows[:, r:r + 1]
        ohhi = (hi_col == iota_h8).astype(jnp.float32)
        dall = jnp.concatenate(
            [f[:, r:r + 1] * ohhi for f in (x1r, y1r, x2r, y2r, vr)], axis=1)
        acc = acc + _dotx(ohlo_t, dall)
    out40[...] = acc


def kernel(anchors, deltas, scores):
    at = jnp.pad(anchors.T, ((0, 0), (0, _NPAD - _N))).reshape(4, _NROWS, _B)
    dt = jnp.pad(deltas.T, ((0, 0), (0, _NPAD - _N))).reshape(4, _NROWS, _B)
    s = jnp.pad(scores, (0, _NPAD - _N))
    planes = [at[i] for i in range(4)]
    planes += [dt[i] for i in range(4)]
    planes.append(s.reshape(_NROWS, _B))
    out40 = pl.pallas_call(
        _main_body,
        out_shape=jax.ShapeDtypeStruct((_B, 40), jnp.float32),
    )(*planes)
    fields = [out40[:, 8 * f:8 * (f + 1)].T.reshape(-1)[:_K2]
              for f in range(5)]
    return jnp.stack(fields, axis=1)


# speedup vs baseline: 1.1709x; 1.0641x over previous
import math

import jax
import jax.numpy as jnp
from jax.experimental import pallas as pl

_N = 20000
_NPAD = 20480
_NROWS = 160
_K1 = 2000
_KPAD = 2048
_NBLK = 16
_B = 128
_K2 = 1000
_T = 0.7
_CLAMP = math.log(1000.0 / 16.0)
_IMG = 1024.0
_NEG = -3.4e38
_IMIN = -2147483648


def _dot(a, b):
    return jax.lax.dot_general(a, b, (((1,), (0,)), ((), ())),
                               preferred_element_type=jnp.float32)


def _dotx(a, b):
    return jax.lax.dot_general(a, b, (((1,), (0,)), ((), ())),
                               preferred_element_type=jnp.float32,
                               precision=jax.lax.Precision.HIGHEST)


def _iota(shape, dim):
    return jax.lax.broadcasted_iota(jnp.int32, shape, dim)


def _exclusive_prefix(m):
    tu = (_iota((_B, _B), 0) <= _iota((_B, _B), 1)).astype(jnp.float32)
    s_r = (_iota((_NROWS, _NROWS), 1) < _iota((_NROWS, _NROWS), 0))
    inc = _dot(m, tu)
    rt = inc[:, _B - 1:_B]
    off = _dot(s_r.astype(jnp.float32), rt)
    return inc - m + off


def _main_body(ax1, ay1, ax2, ay2, dx, dy, dw, dh, sc, out40):
    w = ax2[...] - ax1[...]
    h = ay2[...] - ay1[...]
    cx = ax1[...] + 0.5 * w
    cy = ay1[...] + 0.5 * h
    ddw = jnp.minimum(dw[...], _CLAMP)
    ddh = jnp.minimum(dh[...], _CLAMP)
    px = dx[...] * w + cx
    py = dy[...] * h + cy
    pw = jnp.exp(ddw) * w
    ph = jnp.exp(ddh) * h
    bx1 = jnp.clip(px - 0.5 * pw, 0.0, _IMG)
    by1 = jnp.clip(py - 0.5 * ph, 0.0, _IMG)
    bx2 = jnp.clip(px + 0.5 * pw, 0.0, _IMG)
    by2 = jnp.clip(py + 0.5 * ph, 0.0, _IMG)
    valid = ((bx2 - bx1) > 0.0) & ((by2 - by1) > 0.0)
    sv = jnp.where(valid, jnp.maximum(sc[...], _NEG), _NEG)

    ib = jax.lax.bitcast_convert_type(sv, jnp.int32)
    key = jnp.where(ib >= 0, ib, ib ^ jnp.int32(0x7FFFFFFF))
    ukey = key ^ jnp.int32(_IMIN)

    def rbody(t, c):
        cand, prefix, r = c
        k = 31 - t
        bit = jax.lax.shift_left(jnp.int32(1), k)
        hasf = jnp.where((ukey & bit) != 0, 1.0, 0.0)
        c1 = jnp.sum(cand * hasf)
        take = c1 >= r
        cand2 = jnp.where(take, cand * hasf, cand * (1.0 - hasf))
        prefix2 = jnp.where(take, prefix | bit, prefix)
        r2 = jnp.where(take, r, r - c1)
        return cand2, prefix2, r2

    ones = jnp.ones((_NROWS, _B), jnp.float32)
    tief, vu, r_fin = jax.lax.fori_loop(
        0, 32, rbody, (ones, jnp.int32(0), jnp.float32(_K1)))
    vk = vu ^ _IMIN
    tie = tief > 0.0

    tie_pre = _exclusive_prefix(tief)
    sel = (key > vk) | (tie & (tie_pre < r_fin))
    self32 = jnp.where(sel, 1.0, 0.0)
    pos = _exclusive_prefix(self32)

    posi = pos.astype(jnp.int32)
    lo = jnp.where(sel, posi & (_B - 1), _B)
    posT = jnp.transpose(pos)
    hiT = posT.astype(jnp.int32) >> 7
    dataT = [jnp.transpose(f) for f in (bx1, by1, bx2, by2, sv)]
    iota_cl = _iota((_B, _B), 0)
    iota_h16 = _iota((_B, _NBLK), 1)
    acc_c = jnp.zeros((_B, 80), jnp.float32)
    for i in range(_NROWS):
        lo_row = lo[i:i + 1, :]
        ohlo_t = (iota_cl == lo_row).astype(jnp.float32)
        ohhi = (hiT[:, i:i + 1] == iota_h16).astype(jnp.float32)
        dall = jnp.concatenate([f[:, i:i + 1] * ohhi for f in dataT], axis=1)
        acc_c = acc_c + _dotx(ohlo_t, dall)

    cfields = [acc_c[:, 16 * f:16 * (f + 1)] for f in range(5)]
    slot_rows = _iota((_B, _NBLK), 0) + _B * _iota((_B, _NBLK), 1)
    csv = jnp.where(slot_rows < _K1, cfields[4], _NEG)

    csv_cols = jnp.reshape(jnp.transpose(csv), (1, _KPAD))
    slot_cols = _iota((1, _KPAD), 1)
    ord_parts = []
    for r in range(_NBLK):
        kb = csv[:, r:r + 1]
        sb = slot_rows[:, r:r + 1]
        gt = jnp.sum((csv_cols > kb).astype(jnp.float32), axis=1,
                     keepdims=True)
        eqlt = jnp.sum(((csv_cols == kb) & (slot_cols < sb)).astype(
            jnp.float32), axis=1, keepdims=True)
        ord_parts.append(gt + eqlt)
    ordr = jnp.concatenate(ord_parts, axis=1)

    ordT = jnp.transpose(ordr)
    acc_s = jnp.zeros((_B, 80), jnp.float32)
    for r in range(_NBLK):
        lo_row = ordT[r:r + 1, :].astype(jnp.int32) & (_B - 1)
        ohlo_t = (iota_cl == lo_row).astype(jnp.float32)
        hi_col = ordr[:, r:r + 1].astype(jnp.int32) >> 7
        ohhi = (hi_col == iota_h16).astype(jnp.float32)
        dall = jnp.concatenate(
            [f[:, r:r + 1] * ohhi for f in cfields], axis=1)
        acc_s = acc_s + _dotx(ohlo_t, dall)
    x1r, y1r, x2r, y2r, vr = [acc_s[:, 16 * f:16 * (f + 1)] for f in range(5)]

    cx1, cy1, cx2, cy2 = [jnp.reshape(jnp.transpose(f), (1, _KPAD))
                          for f in (x1r, y1r, x2r, y2r)]
    area_c = (cx2 - cx1) * (cy2 - cy1)
    colg = _iota((1, _KPAD), 1)
    keep = colg < _K1

    upper = _iota((_B, _B), 0) < _iota((_B, _B), 1)

    for r in range(_NBLK):
        base = r * _B
        rx1 = x1r[:, r:r + 1]
        ry1 = y1r[:, r:r + 1]
        rx2 = x2r[:, r:r + 1]
        ry2 = y2r[:, r:r + 1]
        area_r = (rx2 - rx1) * (ry2 - ry1)
        wx = jnp.maximum(jnp.minimum(rx2, cx2[:, base:]) -
                         jnp.maximum(rx1, cx1[:, base:]), 0.0)
        wy = jnp.maximum(jnp.minimum(ry2, cy2[:, base:]) -
                         jnp.maximum(ry1, cy1[:, base:]), 0.0)
        inter = wx * wy
        union = area_r + area_c[:, base:] - inter
        mf = (inter > _T * jnp.maximum(union, 1e-9)).astype(jnp.float32)
        mbb = jnp.where(upper, mf[:, :_B], 0.0)

        active0 = keep[:, base:base + _B].astype(jnp.float32)

        def cond(c):
            return jnp.logical_not(c[2])

        def body(c):
            act, _, _ = c
            pse = _dot(act, mbb) > 0.0
            knew = jnp.where(pse, 0.0, act)
            ns = _dot(knew, mbb) > 0.0
            act2 = jnp.where(ns, 0.0, act)
            done = jnp.all(act2 == act)
            return act2, knew, done

        _, kept, _ = jax.lax.while_loop(
            cond, body, (active0, active0, jnp.bool_(False)))

        ntail = _KPAD - base
        local = _iota((1, ntail), 1)
        supp_tail = (_dot(kept, mf) > 0.0) & (local >= _B)
        su_blk = (active0 > 0.0) & jnp.logical_not(kept > 0.0)
        if r < _NBLK - 1:
            su_blk = jnp.concatenate(
                [su_blk, jnp.zeros((1, ntail - _B), jnp.bool_)], axis=1)
        su_tail = supp_tail | su_blk
        if r > 0:
            su_tail = jnp.concatenate(
                [jnp.zeros((1, base), jnp.bool_), su_tail], axis=1)
        keep = keep & jnp.logical_not(su_tail)

    kept16 = jnp.reshape(keep.astype(jnp.float32), (_NBLK, _B))
    tu = (_iota((_B, _B), 0) <= _iota((_B, _B), 1)).astype(jnp.float32)
    s16 = (_iota((_NBLK, _NBLK), 1) < _iota((_NBLK, _NBLK), 0)).astype(
        jnp.float32)

    def _explusive_prefix16(m16):
        inc = _dot(m16, tu)
        rt = inc[:, _B - 1:_B]
        off = _dot(s16, rt)
        return inc - m16 + off

    exk = _explusive_prefix16(kept16)
    nk16 = 1.0 - kept16
    exn = _explusive_prefix16(nk16)
    ktot = _dot(jnp.ones((1, _NBLK), jnp.float32),
                _dot(kept16, tu)[:, _B - 1:_B])
    pos2f = jnp.where(kept16 > 0.0, exk, ktot + exn)
    pos2 = pos2f.astype(jnp.int32)
    lo16 = pos2 & (_B - 1)
    valid16 = pos2 < 1024
    hi_rows = jnp.transpose(pos2f).astype(jnp.int32) >> 7

    iota_h8 = _iota((_B, 8), 1)
    acc = jnp.zeros((_B, 40), jnp.float32)
    for r in range(_NBLK):
        lo_row = lo16[r:r + 1, :]
        v_row = valid16[r:r + 1, :]
        ohlo_t = ((iota_cl == lo_row) & v_row).astype(jnp.float32)
        hi_col = hi_rows[:, r:r + 1]
        ohhi = (hi_col == iota_h8).astype(jnp.float32)
        dall = jnp.concatenate(
            [f[:, r:r + 1] * ohhi for f in (x1r, y1r, x2r, y2r, vr)], axis=1)
        acc = acc + _dotx(ohlo_t, dall)

    rows8 = [jnp.reshape(jnp.transpose(acc[:, 8 * f:8 * (f + 1)]), (1, 1024))
             for f in range(5)]
    stack_t = jnp.concatenate(rows8 + [jnp.zeros((3, 1024), jnp.float32)],
                              axis=0)
    out40[...] = jnp.transpose(stack_t)


def kernel(anchors, deltas, scores):
    at = jnp.pad(anchors.T, ((0, 0), (0, _NPAD - _N))).reshape(4, _NROWS, _B)
    dt = jnp.pad(deltas.T, ((0, 0), (0, _NPAD - _N))).reshape(4, _NROWS, _B)
    s = jnp.pad(scores, (0, _NPAD - _N))
    planes = [at[i] for i in range(4)]
    planes += [dt[i] for i in range(4)]
    planes.append(s.reshape(_NROWS, _B))
    out1024 = pl.pallas_call(
        _main_body,
        out_shape=jax.ShapeDtypeStruct((1024, 8), jnp.float32),
    )(*planes)
    return out1024[:_K2, :5]


# speedup vs baseline: 1.1726x; 1.0015x over previous
import math

import jax
import jax.numpy as jnp
from jax.experimental import pallas as pl

_N = 20000
_NPAD = 20480
_NROWS = 160
_K1 = 2000
_KPAD = 2048
_NBLK = 16
_B = 128
_K2 = 1000
_T = 0.7
_CLAMP = math.log(1000.0 / 16.0)
_IMG = 1024.0
_NEG = -3.4e38
_IMIN = -2147483648


def _dot(a, b):
    return jax.lax.dot_general(a, b, (((1,), (0,)), ((), ())),
                               preferred_element_type=jnp.float32)


def _dotx(a, b):
    return jax.lax.dot_general(a, b, (((1,), (0,)), ((), ())),
                               preferred_element_type=jnp.float32,
                               precision=jax.lax.Precision.HIGHEST)


def _iota(shape, dim):
    return jax.lax.broadcasted_iota(jnp.int32, shape, dim)


def _exclusive_prefix(m):
    tu = (_iota((_B, _B), 0) <= _iota((_B, _B), 1)).astype(jnp.float32)
    s_r = (_iota((_NROWS, _NROWS), 1) < _iota((_NROWS, _NROWS), 0))
    inc = _dot(m, tu)
    rt = inc[:, _B - 1:_B]
    off = _dot(s_r.astype(jnp.float32), rt)
    return inc - m + off


def _main_body(ax1, ay1, ax2, ay2, dx, dy, dw, dh, sc, out40):
    w = ax2[...] - ax1[...]
    h = ay2[...] - ay1[...]
    cx = ax1[...] + 0.5 * w
    cy = ay1[...] + 0.5 * h
    ddw = jnp.minimum(dw[...], _CLAMP)
    ddh = jnp.minimum(dh[...], _CLAMP)
    px = dx[...] * w + cx
    py = dy[...] * h + cy
    pw = jnp.exp(ddw) * w
    ph = jnp.exp(ddh) * h
    bx1 = jnp.clip(px - 0.5 * pw, 0.0, _IMG)
    by1 = jnp.clip(py - 0.5 * ph, 0.0, _IMG)
    bx2 = jnp.clip(px + 0.5 * pw, 0.0, _IMG)
    by2 = jnp.clip(py + 0.5 * ph, 0.0, _IMG)
    valid = ((bx2 - bx1) > 0.0) & ((by2 - by1) > 0.0)
    sv = jnp.where(valid, jnp.maximum(sc[...], _NEG), _NEG)

    ib = jax.lax.bitcast_convert_type(sv, jnp.int32)
    key = jnp.where(ib >= 0, ib, ib ^ jnp.int32(0x7FFFFFFF))
    ukey = key ^ jnp.int32(_IMIN)

    def rbody(t, c):
        cand, prefix, r = c
        k = 31 - t
        bit = jax.lax.shift_left(jnp.int32(1), k)
        hasf = jnp.where((ukey & bit) != 0, 1.0, 0.0)
        c1 = jnp.sum(cand * hasf)
        take = c1 >= r
        cand2 = jnp.where(take, cand * hasf, cand * (1.0 - hasf))
        prefix2 = jnp.where(take, prefix | bit, prefix)
        r2 = jnp.where(take, r, r - c1)
        return cand2, prefix2, r2

    ones = jnp.ones((_NROWS, _B), jnp.float32)
    tief, vu, r_fin = jax.lax.fori_loop(
        0, 32, rbody, (ones, jnp.int32(0), jnp.float32(_K1)))
    vk = vu ^ _IMIN
    tie = tief > 0.0

    tie_pre = _exclusive_prefix(tief)
    sel = (key > vk) | (tie & (tie_pre < r_fin))
    self32 = jnp.where(sel, 1.0, 0.0)
    pos = _exclusive_prefix(self32)

    posi = pos.astype(jnp.int32)
    lo = jnp.where(sel, posi & (_B - 1), _B)
    posT = jnp.transpose(pos)
    hiT = posT.astype(jnp.int32) >> 7
    dataT = [jnp.transpose(f) for f in (bx1, by1, bx2, by2, sv)]
    iota_cl = _iota((_B, _B), 0)
    iota_h16 = _iota((_B, _NBLK), 1)
    acc_c = jnp.zeros((_B, 80), jnp.float32)
    for i in range(_NROWS):
        lo_row = lo[i:i + 1, :]
        ohlo_t = (iota_cl == lo_row).astype(jnp.float32)
        ohhi = (hiT[:, i:i + 1] == iota_h16).astype(jnp.float32)
        dall = jnp.concatenate([f[:, i:i + 1] * ohhi for f in dataT], axis=1)
        acc_c = acc_c + _dotx(ohlo_t, dall)

    cfields = [acc_c[:, 16 * f:16 * (f + 1)] for f in range(5)]
    slot_rows = _iota((_B, _NBLK), 0) + _B * _iota((_B, _NBLK), 1)
    csv = jnp.where(slot_rows < _K1, cfields[4], _NEG)

    csv_cols = jnp.reshape(jnp.transpose(csv), (1, _KPAD))
    slot_cols = _iota((1, _KPAD), 1)
    ord_parts = []
    for r in range(_NBLK):
        kb = csv[:, r:r + 1]
        sb = slot_rows[:, r:r + 1]
        gt = jnp.sum((csv_cols > kb).astype(jnp.float32), axis=1,
                     keepdims=True)
        eqlt = jnp.sum(((csv_cols == kb) & (slot_cols < sb)).astype(
            jnp.float32), axis=1, keepdims=True)
        ord_parts.append(gt + eqlt)
    ordr = jnp.concatenate(ord_parts, axis=1)

    ordT = jnp.transpose(ordr)
    acc_s = jnp.zeros((_B, 80), jnp.float32)
    for r in range(_NBLK):
        lo_row = ordT[r:r + 1, :].astype(jnp.int32) & (_B - 1)
        ohlo_t = (iota_cl == lo_row).astype(jnp.float32)
        hi_col = ordr[:, r:r + 1].astype(jnp.int32) >> 7
        ohhi = (hi_col == iota_h16).astype(jnp.float32)
        dall = jnp.concatenate(
            [f[:, r:r + 1] * ohhi for f in cfields], axis=1)
        acc_s = acc_s + _dotx(ohlo_t, dall)
    x1r, y1r, x2r, y2r, vr = [acc_s[:, 16 * f:16 * (f + 1)] for f in range(5)]

    cx1, cy1, cx2, cy2 = [jnp.reshape(jnp.transpose(f), (1, _KPAD))
                          for f in (x1r, y1r, x2r, y2r)]
    area_c = (cx2 - cx1) * (cy2 - cy1)
    colg = _iota((1, _KPAD), 1)
    keep = colg < _K1

    upper = _iota((_B, _B), 0) < _iota((_B, _B), 1)

    for r in range(_NBLK):
        base = r * _B
        rx1 = x1r[:, r:r + 1]
        ry1 = y1r[:, r:r + 1]
        rx2 = x2r[:, r:r + 1]
        ry2 = y2r[:, r:r + 1]
        area_r = (rx2 - rx1) * (ry2 - ry1)
        wx = jnp.maximum(jnp.minimum(rx2, cx2[:, base:]) -
                         jnp.maximum(rx1, cx1[:, base:]), 0.0)
        wy = jnp.maximum(jnp.minimum(ry2, cy2[:, base:]) -
                         jnp.maximum(ry1, cy1[:, base:]), 0.0)
        inter = wx * wy
        union = area_r + area_c[:, base:] - inter
        iou = inter / jnp.maximum(union, 1e-9)
        mf = (iou > _T).astype(jnp.float32)
        mbb = jnp.where(upper, mf[:, :_B], 0.0)

        active0 = keep[:, base:base + _B].astype(jnp.float32)

        def cond(c):
            return jnp.logical_not(c[2])

        def body(c):
            act, _, _ = c
            pse = _dot(act, mbb) > 0.0
            knew = jnp.where(pse, 0.0, act)
            ns = _dot(knew, mbb) > 0.0
            act2 = jnp.where(ns, 0.0, act)
            done = jnp.all(act2 == act)
            return act2, knew, done

        _, kept, _ = jax.lax.while_loop(
            cond, body, (active0, active0, jnp.bool_(False)))

        ntail = _KPAD - base
        local = _iota((1, ntail), 1)
        supp_tail = (_dot(kept, mf) > 0.0) & (local >= _B)
        su_blk = (active0 > 0.0) & jnp.logical_not(kept > 0.0)
        if r < _NBLK - 1:
            su_blk = jnp.concatenate(
                [su_blk, jnp.zeros((1, ntail - _B), jnp.bool_)], axis=1)
        su_tail = supp_tail | su_blk
        if r > 0:
            su_tail = jnp.concatenate(
                [jnp.zeros((1, base), jnp.bool_), su_tail], axis=1)
        keep = keep & jnp.logical_not(su_tail)

    kept16 = jnp.reshape(keep.astype(jnp.float32), (_NBLK, _B))
    tu = (_iota((_B, _B), 0) <= _iota((_B, _B), 1)).astype(jnp.float32)
    s16 = (_iota((_NBLK, _NBLK), 1) < _iota((_NBLK, _NBLK), 0)).astype(
        jnp.float32)

    def _explusive_prefix16(m16):
        inc = _dot(m16, tu)
        rt = inc[:, _B - 1:_B]
        off = _dot(s16, rt)
        return inc - m16 + off

    exk = _explusive_prefix16(kept16)
    nk16 = 1.0 - kept16
    exn = _explusive_prefix16(nk16)
    ktot = _dot(jnp.ones((1, _NBLK), jnp.float32),
                _dot(kept16, tu)[:, _B - 1:_B])
    pos2f = jnp.where(kept16 > 0.0, exk, ktot + exn)
    pos2 = pos2f.astype(jnp.int32)
    lo16 = pos2 & (_B - 1)
    valid16 = pos2 < 1024
    hi_rows = jnp.transpose(pos2f).astype(jnp.int32) >> 7

    iota_h8 = _iota((_B, 8), 1)
    acc = jnp.zeros((_B, 40), jnp.float32)
    for r in range(_NBLK):
        lo_row = lo16[r:r + 1, :]
        v_row = valid16[r:r + 1, :]
        ohlo_t = ((iota_cl == lo_row) & v_row).astype(jnp.float32)
        hi_col = hi_rows[:, r:r + 1]
        ohhi = (hi_col == iota_h8).astype(jnp.float32)
        dall = jnp.concatenate(
            [f[:, r:r + 1] * ohhi for f in (x1r, y1r, x2r, y2r, vr)], axis=1)
        acc = acc + _dotx(ohlo_t, dall)

    rows8 = [jnp.reshape(jnp.transpose(acc[:, 8 * f:8 * (f + 1)]), (1, 1024))
             for f in range(5)]
    stack_t = jnp.concatenate(rows8 + [jnp.zeros((3, 1024), jnp.float32)],
                              axis=0)
    out40[...] = jnp.transpose(stack_t)


def kernel(anchors, deltas, scores):
    at = jnp.pad(anchors.T, ((0, 0), (0, _NPAD - _N))).reshape(4, _NROWS, _B)
    dt = jnp.pad(deltas.T, ((0, 0), (0, _NPAD - _N))).reshape(4, _NROWS, _B)
    s = jnp.pad(scores, (0, _NPAD - _N))
    planes = [at[i] for i in range(4)]
    planes += [dt[i] for i in range(4)]
    planes.append(s.reshape(_NROWS, _B))
    out1024 = pl.pallas_call(
        _main_body,
        out_shape=jax.ShapeDtypeStruct((1024, 8), jnp.float32),
    )(*planes)
    return out1024[:_K2, :5]


# speedup vs baseline: 1.7941x; 1.5300x over previous
import math

import jax
import jax.numpy as jnp
from jax.experimental import pallas as pl

_N = 20000
_NPAD = 20480
_NROWS = 160
_K1 = 2000
_KPAD = 2048
_NBLK = 16
_B = 128
_K2 = 1000
_T = 0.7
_CLAMP = math.log(1000.0 / 16.0)
_IMG = 1024.0
_NEG = -3.4e38
_IMIN = -2147483648


def _dot(a, b):
    return jax.lax.dot_general(a, b, (((1,), (0,)), ((), ())),
                               preferred_element_type=jnp.float32)


def _dotx(a, b):
    return jax.lax.dot_general(a, b, (((1,), (0,)), ((), ())),
                               preferred_element_type=jnp.float32,
                               precision=jax.lax.Precision.HIGHEST)


def _iota(shape, dim):
    return jax.lax.broadcasted_iota(jnp.int32, shape, dim)


def _exclusive_prefix(m):
    tu = (_iota((_B, _B), 0) <= _iota((_B, _B), 1)).astype(jnp.float32)
    s_r = (_iota((_NROWS, _NROWS), 1) < _iota((_NROWS, _NROWS), 0))
    inc = _dot(m, tu)
    rt = inc[:, _B - 1:_B]
    off = _dot(s_r.astype(jnp.float32), rt)
    return inc - m + off


def _main_body(ax1, ay1, ax2, ay2, dx, dy, dw, dh, sc, out40):
    w = ax2[...] - ax1[...]
    h = ay2[...] - ay1[...]
    cx = ax1[...] + 0.5 * w
    cy = ay1[...] + 0.5 * h
    ddw = jnp.minimum(dw[...], _CLAMP)
    ddh = jnp.minimum(dh[...], _CLAMP)
    px = dx[...] * w + cx
    py = dy[...] * h + cy
    pw = jnp.exp(ddw) * w
    ph = jnp.exp(ddh) * h
    bx1 = jnp.clip(px - 0.5 * pw, 0.0, _IMG)
    by1 = jnp.clip(py - 0.5 * ph, 0.0, _IMG)
    bx2 = jnp.clip(px + 0.5 * pw, 0.0, _IMG)
    by2 = jnp.clip(py + 0.5 * ph, 0.0, _IMG)
    valid = ((bx2 - bx1) > 0.0) & ((by2 - by1) > 0.0)
    sv = jnp.where(valid, jnp.maximum(sc[...], _NEG), _NEG)

    ib = jax.lax.bitcast_convert_type(sv, jnp.int32)
    key = jnp.where(ib >= 0, ib, ib ^ jnp.int32(0x7FFFFFFF))
    ukey = key ^ jnp.int32(_IMIN)

    def rbody(t, c):
        cand, prefix, r = c
        k = 31 - t
        bit = jax.lax.shift_left(jnp.int32(1), k)
        hasf = jnp.where((ukey & bit) != 0, 1.0, 0.0)
        c1 = jnp.sum(cand * hasf)
        take = c1 >= r
        cand2 = jnp.where(take, cand * hasf, cand * (1.0 - hasf))
        prefix2 = jnp.where(take, prefix | bit, prefix)
        r2 = jnp.where(take, r, r - c1)
        return cand2, prefix2, r2

    ones = jnp.ones((_NROWS, _B), jnp.float32)
    tief, vu, r_fin = jax.lax.fori_loop(
        0, 32, rbody, (ones, jnp.int32(0), jnp.float32(_K1)))
    vk = vu ^ _IMIN
    tie = tief > 0.0

    tie_pre = _exclusive_prefix(tief)
    sel = (key > vk) | (tie & (tie_pre < r_fin))
    self32 = jnp.where(sel, 1.0, 0.0)
    pos = _exclusive_prefix(self32)

    posi = pos.astype(jnp.int32)
    lo = jnp.where(sel, posi & (_B - 1), _B)
    hi = posi >> 7
    loT = jnp.transpose(lo)
    iota_cl_row = _iota((1, _B), 1)
    iota_h16c = _iota((_NBLK, _B), 0)
    acc_c = jnp.zeros((80, _B), jnp.float32)
    for i in range(_NROWS):
        ohlo = (loT[:, i:i + 1] == iota_cl_row).astype(jnp.float32)
        ohhi_t = (hi[i:i + 1, :] == iota_h16c).astype(jnp.float32)
        dall_t = jnp.concatenate(
            [f[i:i + 1, :] * ohhi_t for f in (bx1, by1, bx2, by2, sv)],
            axis=0)
        acc_c = acc_c + _dotx(dall_t, ohlo)

    cfields_t = [acc_c[16 * f:16 * (f + 1), :] for f in range(5)]
    slot_t = _B * _iota((_NBLK, _B), 0) + _iota((_NBLK, _B), 1)
    csv_t = jnp.where(slot_t < _K1, cfields_t[4], _NEG)

    csv_cols = jnp.reshape(csv_t, (1, _KPAD))
    csv_rows = jnp.transpose(csv_t)
    slot_rows = _iota((_B, _NBLK), 0) + _B * _iota((_B, _NBLK), 1)
    slot_cols = _iota((1, _KPAD), 1)
    ord_parts = []
    for r in range(_NBLK):
        kb = csv_rows[:, r:r + 1]
        sb = slot_rows[:, r:r + 1]
        gt = jnp.sum((csv_cols > kb).astype(jnp.float32), axis=1,
                     keepdims=True)
        eqlt = jnp.sum(((csv_cols == kb) & (slot_cols < sb)).astype(
            jnp.float32), axis=1, keepdims=True)
        ord_parts.append(gt + eqlt)
    ordr = jnp.concatenate(ord_parts, axis=1)
    ordi = ordr.astype(jnp.int32)
    ord_lo = ordi & (_B - 1)
    ordT = jnp.transpose(ordi)
    ord_hi_t = ordT >> 7

    acc_s = jnp.zeros((80, _B), jnp.float32)
    for r in range(_NBLK):
        ohlo = (ord_lo[:, r:r + 1] == iota_cl_row).astype(jnp.float32)
        ohhi_t = (ord_hi_t[r:r + 1, :] == iota_h16c).astype(jnp.float32)
        dall_t = jnp.concatenate(
            [f[r:r + 1, :] * ohhi_t for f in cfields_t], axis=0)
        acc_s = acc_s + _dotx(dall_t, ohlo)
    sfields_t = [acc_s[16 * f:16 * (f + 1), :] for f in range(5)]
    x1r, y1r, x2r, y2r = [jnp.transpose(f) for f in sfields_t[:4]]

    cx1, cy1, cx2, cy2 = [jnp.reshape(f, (1, _KPAD)) for f in sfields_t[:4]]
    area_c = (cx2 - cx1) * (cy2 - cy1)
    colg = _iota((1, _KPAD), 1)
    keep = colg < _K1

    upper = _iota((_B, _B), 0) < _iota((_B, _B), 1)

    for r in range(_NBLK):
        base = r * _B
        rx1 = x1r[:, r:r + 1]
        ry1 = y1r[:, r:r + 1]
        rx2 = x2r[:, r:r + 1]
        ry2 = y2r[:, r:r + 1]
        area_r = (rx2 - rx1) * (ry2 - ry1)
        wx = jnp.maximum(jnp.minimum(rx2, cx2[:, base:]) -
                         jnp.maximum(rx1, cx1[:, base:]), 0.0)
        wy = jnp.maximum(jnp.minimum(ry2, cy2[:, base:]) -
                         jnp.maximum(ry1, cy1[:, base:]), 0.0)
        inter = wx * wy
        union = area_r + area_c[:, base:] - inter
        iou = inter / jnp.maximum(union, 1e-9)
        mf = (iou > _T).astype(jnp.float32)
        mbb = jnp.where(upper, mf[:, :_B], 0.0)

        active0 = keep[:, base:base + _B].astype(jnp.float32)

        def cond(c):
            return jnp.logical_not(c[2])

        def body(c):
            act, _, _ = c
            pse = _dot(act, mbb) > 0.0
            knew = jnp.where(pse, 0.0, act)
            ns = _dot(knew, mbb) > 0.0
            act2 = jnp.where(ns, 0.0, act)
            done = jnp.all(act2 == act)
            return act2, knew, done

        _, kept, _ = jax.lax.while_loop(
            cond, body, (active0, active0, jnp.bool_(False)))

        ntail = _KPAD - base
        local = _iota((1, ntail), 1)
        supp_tail = (_dot(kept, mf) > 0.0) & (local >= _B)
        su_blk = (active0 > 0.0) & jnp.logical_not(kept > 0.0)
        if r < _NBLK - 1:
            su_blk = jnp.concatenate(
                [su_blk, jnp.zeros((1, ntail - _B), jnp.bool_)], axis=1)
        su_tail = supp_tail | su_blk
        if r > 0:
            su_tail = jnp.concatenate(
                [jnp.zeros((1, base), jnp.bool_), su_tail], axis=1)
        keep = keep & jnp.logical_not(su_tail)

    kept16 = jnp.reshape(keep.astype(jnp.float32), (_NBLK, _B))
    tu = (_iota((_B, _B), 0) <= _iota((_B, _B), 1)).astype(jnp.float32)
    s16 = (_iota((_NBLK, _NBLK), 1) < _iota((_NBLK, _NBLK), 0)).astype(
        jnp.float32)

    def _explusive_prefix16(m16):
        inc = _dot(m16, tu)
        rt = inc[:, _B - 1:_B]
        off = _dot(s16, rt)
        return inc - m16 + off

    exk = _explusive_prefix16(kept16)
    nk16 = 1.0 - kept16
    exn = _explusive_prefix16(nk16)
    ktot = _dot(jnp.ones((1, _NBLK), jnp.float32),
                _dot(kept16, tu)[:, _B - 1:_B])
    pos2f = jnp.where(kept16 > 0.0, exk, ktot + exn)
    pos2 = pos2f.astype(jnp.int32)
    pos2t = jnp.transpose(pos2f).astype(jnp.int32)
    lo_f = pos2t & (_B - 1)
    valid_f = pos2t < 1024

    iota_h8c = _iota((8, _B), 0)
    acc_f = jnp.zeros((40, _B), jnp.float32)
    for r in range(_NBLK):
        ohlo = ((lo_f[:, r:r + 1] == iota_cl_row)
                & valid_f[:, r:r + 1]).astype(jnp.float32)
        hi_row = pos2[r:r + 1, :] >> 7
        ohhi_t = (hi_row == iota_h8c).astype(jnp.float32)
        dall_t = jnp.concatenate(
            [f[r:r + 1, :] * ohhi_t for f in sfields_t], axis=0)
        acc_f = acc_f + _dotx(dall_t, ohlo)

    rows8 = [jnp.reshape(acc_f[8 * f:8 * (f + 1), :], (1, 1024))
             for f in range(5)]
    stack_t = jnp.concatenate(rows8 + [jnp.zeros((3, 1024), jnp.float32)],
                              axis=0)
    out40[...] = jnp.transpose(stack_t)


def kernel(anchors, deltas, scores):
    at = jnp.pad(anchors.T, ((0, 0), (0, _NPAD - _N))).reshape(4, _NROWS, _B)
    dt = jnp.pad(deltas.T, ((0, 0), (0, _NPAD - _N))).reshape(4, _NROWS, _B)
    s = jnp.pad(scores, (0, _NPAD - _N))
    planes = [at[i] for i in range(4)]
    planes += [dt[i] for i in range(4)]
    planes.append(s.reshape(_NROWS, _B))
    out1024 = pl.pallas_call(
        _main_body,
        out_shape=jax.ShapeDtypeStruct((1024, 8), jnp.float32),
    )(*planes)
    return out1024[:_K2, :5]
